# Initial kernel scaffold; baseline (speedup 1.0000x reference)
import functools
import math

import jax
import jax.numpy as jnp
from jax import lax
from jax.experimental import pallas as pl
from jax.experimental.pallas import tpu as pltpu
from jax.experimental.pallas import tpu_sc as plsc

BSZ, NODES, EDGES, D = 8, 512, 2048, 512
N_TOTAL = BSZ * NODES
E_TOTAL = BSZ * EDGES
G_TOTAL = 2 * E_TOTAL

_PROJ_BLK = 512


def _proj_body(x_ref, w_ref, out_ref):
    out_ref[...] = jnp.dot(x_ref[...], w_ref[0],
                           preferred_element_type=jnp.float32)


def _project_nodes(x, w_stack):
    nblk = N_TOTAL // _PROJ_BLK
    return pl.pallas_call(
        _proj_body,
        grid=(2 * nblk,),
        in_specs=[
            pl.BlockSpec((_PROJ_BLK, D), lambda j: (j % nblk, 0)),
            pl.BlockSpec((1, D, D), lambda j: (j // nblk, 0, 0)),
        ],
        out_specs=pl.BlockSpec((_PROJ_BLK, D), lambda j: (j, 0)),
        out_shape=jax.ShapeDtypeStruct((2 * N_TOTAL, D), jnp.float32),
    )(x, w_stack)


_SC_INFO = plsc.get_sparse_core_info()
_NC, _NS = _SC_INFO.num_cores, _SC_INFO.num_subcores
_NW = _NC * _NS
_PER_TILE = G_TOTAL // _NW
_CHUNK = 128
_NCHUNK = _PER_TILE // _CHUNK

_sc_mesh = plsc.VectorSubcoreMesh(core_axis_name="c", subcore_axis_name="s")


@functools.partial(
    pl.kernel,
    out_type=jax.ShapeDtypeStruct((G_TOTAL, D), jnp.float32),
    mesh=_sc_mesh,
    scratch_types=[
        pltpu.VMEM((_CHUNK,), jnp.int32),
        pltpu.VMEM((_CHUNK, D), jnp.float32),
        pltpu.SemaphoreType.DMA,
    ],
)
def _sc_gather(tbl_hbm, idx_hbm, out_hbm, idx_v, rows_v, sem):
    wid = lax.axis_index("s") * _NC + lax.axis_index("c")
    base = wid * _PER_TILE
    for c in range(_NCHUNK):
        off = base + c * _CHUNK
        pltpu.sync_copy(idx_hbm.at[pl.ds(off, _CHUNK)], idx_v)
        pltpu.async_copy(tbl_hbm.at[idx_v], rows_v, sem).wait()
        pltpu.sync_copy(rows_v, out_hbm.at[pl.ds(off, _CHUNK)])


_MLP_BLK = 1024
_INV_SQRT2 = 1.0 / math.sqrt(2.0)


def _mlp_body(gh_ref, gt_ref, r_ref, w1r_ref, b1_ref, gamma_ref, beta_ref,
              w2_ref, b2_ref, out_ref):
    m = jnp.dot(r_ref[...], w1r_ref[...], preferred_element_type=jnp.float32)
    h1 = m + gh_ref[...] + gt_ref[...] + b1_ref[...]
    mu = jnp.mean(h1, axis=-1, keepdims=True)
    var = jnp.mean((h1 - mu) ** 2, axis=-1, keepdims=True)
    h1n = (h1 - mu) * lax.rsqrt(var + 1e-5) * gamma_ref[...] + beta_ref[...]
    h1a = h1n * 0.5 * (1.0 + lax.erf(h1n * _INV_SQRT2))
    out_ref[...] = jnp.dot(h1a, w2_ref[...],
                           preferred_element_type=jnp.float32) + b2_ref[...]


def _mlp(g, edge_rep, w1r, b1, gamma, beta, w2, b2):
    eblk = E_TOTAL // _MLP_BLK
    vec = pl.BlockSpec((1, D), lambda j: (0, 0))
    mat = pl.BlockSpec((D, D), lambda j: (0, 0))
    return pl.pallas_call(
        _mlp_body,
        grid=(eblk,),
        in_specs=[
            pl.BlockSpec((_MLP_BLK, D), lambda j: (j, 0)),
            pl.BlockSpec((_MLP_BLK, D), lambda j: (j + eblk, 0)),
            pl.BlockSpec((_MLP_BLK, D), lambda j: (j, 0)),
            mat, vec, vec, vec, mat, vec,
        ],
        out_specs=pl.BlockSpec((_MLP_BLK, D), lambda j: (j, 0)),
        out_shape=jax.ShapeDtypeStruct((E_TOTAL, D), jnp.float32),
    )(g, g, edge_rep, w1r, b1.reshape(1, D), gamma.reshape(1, D),
      beta.reshape(1, D), w2, b2.reshape(1, D))


def kernel(x, batch, edge_index, edge_rep, num_edges, ptr, W1, b1, gamma,
           beta, W2, b2):
    w_stack = jnp.stack([W1[:D], W1[2 * D:]])
    w1r = W1[D:2 * D]
    tbl = _project_nodes(x, w_stack)
    idx = jnp.concatenate([edge_index[0], edge_index[1] + N_TOTAL])
    g = _sc_gather(tbl, idx)
    out = _mlp(g, edge_rep, w1r, b1, gamma, beta, W2, b2)
    mask = jnp.ones((BSZ, EDGES), dtype=jnp.float32)
    return out.reshape(BSZ, EDGES, D), mask


# speedup vs baseline: 5.5899x; 5.5899x over previous
import functools
import math

import jax
import jax.numpy as jnp
from jax import lax
from jax.experimental import pallas as pl
from jax.experimental.pallas import tpu as pltpu
from jax.experimental.pallas import tpu_sc as plsc

BSZ, NODES, EDGES, D = 8, 512, 2048, 512
N_TOTAL = BSZ * NODES
E_TOTAL = BSZ * EDGES
G_TOTAL = 2 * E_TOTAL

_PROJ_BLK = 512


def _proj_body(x_ref, w_ref, out_ref):
    out_ref[...] = jnp.dot(x_ref[...], w_ref[0],
                           preferred_element_type=jnp.float32)


def _project_nodes(x, w_stack):
    nblk = N_TOTAL // _PROJ_BLK
    return pl.pallas_call(
        _proj_body,
        grid=(2 * nblk,),
        in_specs=[
            pl.BlockSpec((_PROJ_BLK, D), lambda j: (j % nblk, 0)),
            pl.BlockSpec((1, D, D), lambda j: (j // nblk, 0, 0)),
        ],
        out_specs=pl.BlockSpec((_PROJ_BLK, D), lambda j: (j, 0)),
        out_shape=jax.ShapeDtypeStruct((2 * N_TOTAL, D), jnp.float32),
    )(x, w_stack)


_NC, _NS = 2, 16
_NW = _NC * _NS
_PER_TILE = G_TOTAL // _NW
_CHUNK = 128
_NCHUNK = _PER_TILE // _CHUNK

@functools.cache
def _sc_gather_kernel():
    mesh = plsc.VectorSubcoreMesh(core_axis_name="c", subcore_axis_name="s",
                                  num_cores=_NC, num_subcores=_NS)

    @functools.partial(
        pl.kernel,
        out_type=jax.ShapeDtypeStruct((G_TOTAL, D), jnp.float32),
        mesh=mesh,
        scratch_types=[
            pltpu.VMEM((_CHUNK,), jnp.int32),
            pltpu.VMEM((_CHUNK, D), jnp.float32),
            pltpu.SemaphoreType.DMA,
        ],
    )
    def body(tbl_hbm, idx_hbm, out_hbm, idx_v, rows_v, sem):
        wid = lax.axis_index("s") * _NC + lax.axis_index("c")
        base = wid * _PER_TILE
        for c in range(_NCHUNK):
            off = base + c * _CHUNK
            pltpu.sync_copy(idx_hbm.at[pl.ds(off, _CHUNK)], idx_v)
            pltpu.async_copy(tbl_hbm.at[idx_v], rows_v, sem).wait()
            pltpu.sync_copy(rows_v, out_hbm.at[pl.ds(off, _CHUNK)])

    return body


def _sc_gather(tbl, idx):
    return _sc_gather_kernel()(tbl, idx)


_MLP_BLK = 1024
_INV_SQRT2 = 1.0 / math.sqrt(2.0)


def _mlp_body(gh_ref, gt_ref, r_ref, w1r_ref, b1_ref, gamma_ref, beta_ref,
              w2_ref, b2_ref, out_ref):
    m = jnp.dot(r_ref[...], w1r_ref[...], preferred_element_type=jnp.float32)
    h1 = m + gh_ref[...] + gt_ref[...] + b1_ref[...]
    mu = jnp.mean(h1, axis=-1, keepdims=True)
    var = jnp.mean((h1 - mu) ** 2, axis=-1, keepdims=True)
    h1n = (h1 - mu) * lax.rsqrt(var + 1e-5) * gamma_ref[...] + beta_ref[...]
    h1a = h1n * 0.5 * (1.0 + lax.erf(h1n * _INV_SQRT2))
    out_ref[...] = jnp.dot(h1a, w2_ref[...],
                           preferred_element_type=jnp.float32) + b2_ref[...]


def _mlp(g, edge_rep, w1r, b1, gamma, beta, w2, b2):
    eblk = E_TOTAL // _MLP_BLK
    vec = pl.BlockSpec((1, D), lambda j: (0, 0))
    mat = pl.BlockSpec((D, D), lambda j: (0, 0))
    return pl.pallas_call(
        _mlp_body,
        grid=(eblk,),
        in_specs=[
            pl.BlockSpec((_MLP_BLK, D), lambda j: (j, 0)),
            pl.BlockSpec((_MLP_BLK, D), lambda j: (j + eblk, 0)),
            pl.BlockSpec((_MLP_BLK, D), lambda j: (j, 0)),
            mat, vec, vec, vec, mat, vec,
        ],
        out_specs=pl.BlockSpec((_MLP_BLK, D), lambda j: (j, 0)),
        out_shape=jax.ShapeDtypeStruct((E_TOTAL, D), jnp.float32),
    )(g, g, edge_rep, w1r, b1.reshape(1, D), gamma.reshape(1, D),
      beta.reshape(1, D), w2, b2.reshape(1, D))


def kernel(x, batch, edge_index, edge_rep, num_edges, ptr, W1, b1, gamma,
           beta, W2, b2):
    w_stack = jnp.stack([W1[:D], W1[2 * D:]])
    w1r = W1[D:2 * D]
    tbl = _project_nodes(x, w_stack)
    idx = jnp.concatenate([edge_index[0], edge_index[1] + N_TOTAL])
    g = _sc_gather(tbl, idx)
    out = _mlp(g, edge_rep, w1r, b1, gamma, beta, W2, b2)
    mask = jnp.ones((BSZ, EDGES), dtype=jnp.float32)
    return out.reshape(BSZ, EDGES, D), mask
